# baseline (speedup 1.0000x reference)
import functools
import math

import jax
import jax.numpy as jnp
from jax import lax
from jax.experimental import pallas as pl
from jax.experimental.pallas import tpu as pltpu
from jax.experimental.pallas import tpu_sc as plsc

ROWS = 4096
SEQ = 50
D = 128
SCALE = math.sqrt(D)

NC = 2
NS = 16
NW = NC * NS
LANES = 16

BAND = ROWS // NW
NBUF = 2


def _body(tok_hbm, table_hbm, out_hbm, idx_v, in_v, out_v, gsem, wsem):
    wid = lax.axis_index("s") * NC + lax.axis_index("c")
    base = wid * BAND

    pltpu.sync_copy(tok_hbm.at[:, pl.ds(base, BAND)], idx_v)

    def gather_start(t, b):
        pltpu.async_copy(table_hbm.at[idx_v.at[t]], in_v[b], gsem[b])

    def gather_wait(t, b):
        pltpu.make_async_copy(table_hbm.at[idx_v.at[t]], in_v[b],
                              gsem[b]).wait()

    def write_start(t, b):
        pltpu.async_copy(out_v[b], out_hbm.at[t, pl.ds(base, BAND)], wsem[b])

    def write_wait(t, b):
        pltpu.make_async_copy(out_v[b], out_hbm.at[t, pl.ds(base, BAND)],
                              wsem[b]).wait()

    def scale(b):
        @pl.loop(0, BAND, unroll=2)
        def _row(r):
            for k in range(D // LANES):
                sl = pl.ds(k * LANES, LANES)
                out_v[b][r, sl] = in_v[b][r, sl] * SCALE

    for b in range(NBUF):
        gather_start(b, b)

    @pl.loop(0, SEQ, step=NBUF)
    def _grp(j):
        for b in range(NBUF):
            t = j + b
            gather_wait(t, b)

            @pl.when(t >= NBUF)
            def _():
                write_wait(t - NBUF, b)

            scale(b)

            @pl.when(t + NBUF < SEQ)
            def _():
                gather_start(t + NBUF, b)

            write_start(t, b)

    for b in range(NBUF):
        write_wait(SEQ - NBUF + b, b)


@jax.jit
def _embed(tokens_t, table):
    mesh = plsc.VectorSubcoreMesh(
        core_axis_name="c", subcore_axis_name="s",
        num_cores=NC, num_subcores=NS,
    )
    kern = pl.kernel(
        _body,
        out_type=jax.ShapeDtypeStruct((SEQ, ROWS, D), jnp.float32),
        mesh=mesh,
        scratch_types=[
            pltpu.VMEM((SEQ, BAND), jnp.int32),
            [pltpu.VMEM((BAND, D), jnp.float32) for _ in range(NBUF)],
            [pltpu.VMEM((BAND, D), jnp.float32) for _ in range(NBUF)],
            [pltpu.SemaphoreType.DMA for _ in range(NBUF)],
            [pltpu.SemaphoreType.DMA for _ in range(NBUF)],
        ],
    )
    return kern(tokens_t, table)


def kernel(tokens, table):
    out = _embed(tokens.astype(jnp.int32).T, table)
    return jnp.swapaxes(out, 0, 1)
